# Initial kernel scaffold; baseline (speedup 1.0000x reference)
import functools

import jax
import jax.numpy as jnp
from jax import lax
from jax.experimental import pallas as pl
from jax.experimental.pallas import tpu as pltpu
from jax.experimental.pallas import tpu_sc as plsc

VOCAB = 1000000
EMB = 32
OUT = 100
B = 16384
L = 50

NC = 2
NS = 16
NW = NC * NS
SENT_PER_W = B // NW
SENT_PER_CHUNK = 2
IDX_PER_CHUNK = SENT_PER_CHUNK * L
CHUNKS_PER_W = SENT_PER_W // SENT_PER_CHUNK
K = 8
NBUF = 2 * K
NPAIR = CHUNKS_PER_W // NBUF
INV_L = 1.0 / L

_sc_mesh = plsc.VectorSubcoreMesh(core_axis_name="c", subcore_axis_name="s")


@functools.partial(
    pl.kernel,
    out_type=jax.ShapeDtypeStruct((B, EMB), jnp.float32),
    mesh=_sc_mesh,
    scratch_types=[
        pltpu.VMEM((CHUNKS_PER_W, IDX_PER_CHUNK), jnp.int32),
        pltpu.VMEM((NBUF, IDX_PER_CHUNK, EMB), jnp.float32),
        pltpu.VMEM((SENT_PER_W, EMB), jnp.float32),
        pltpu.SemaphoreType.DMA,
        pltpu.SemaphoreType.DMA,
    ],
)
def _sc_pool(idx_hbm, table_hbm, out_hbm, idx_v, rows_v, out_v, sem0, sem1):
    wid = lax.axis_index("s") * NC + lax.axis_index("c")

    pltpu.sync_copy(idx_hbm.at[wid], idx_v)

    def fire(j, buf, sem):
        pltpu.make_async_copy(
            table_hbm.at[idx_v.at[j]], rows_v.at[buf], sem).start()

    def drain(buf, sem):
        pltpu.make_async_copy(
            table_hbm.at[idx_v.at[0]], rows_v.at[buf], sem).wait()

    def process(buf, sent_base):
        def acc_body(l, carry):
            a0, a1, b0, b1 = carry
            r = 5 * l
            for u in range(5):
                a0 = a0 + rows_v[buf, r + u, pl.ds(0, 16)]
                a1 = a1 + rows_v[buf, r + u, pl.ds(16, 16)]
                b0 = b0 + rows_v[buf, L + r + u, pl.ds(0, 16)]
                b1 = b1 + rows_v[buf, L + r + u, pl.ds(16, 16)]
            return a0, a1, b0, b1

        z = jnp.zeros((16,), jnp.float32)
        a0, a1, b0, b1 = lax.fori_loop(0, L // 5, acc_body, (z, z, z, z))
        out_v[sent_base, pl.ds(0, 16)] = a0 * INV_L
        out_v[sent_base, pl.ds(16, 16)] = a1 * INV_L
        out_v[sent_base + 1, pl.ds(0, 16)] = b0 * INV_L
        out_v[sent_base + 1, pl.ds(16, 16)] = b1 * INV_L

    for b in range(K):
        fire(b, b, sem0)

    def pair_body(pair, _):
        base = pair * NBUF
        for b in range(K):
            fire(base + K + b, K + b, sem1)
        for b in range(K):
            drain(b, sem0)
        for b in range(K):
            process(b, (base + b) * SENT_PER_CHUNK)

        @pl.when(pair < NPAIR - 1)
        def _():
            for b in range(K):
                fire(base + NBUF + b, b, sem0)

        for b in range(K):
            drain(K + b, sem1)
        for b in range(K):
            process(K + b, (base + K + b) * SENT_PER_CHUNK)
        return 0

    lax.fori_loop(0, NPAIR, pair_body, 0)

    pltpu.sync_copy(out_v, out_hbm.at[pl.ds(wid * SENT_PER_W, SENT_PER_W)])


BT = 2048
OUT_PAD = 128
NEG = -1e30


def _tc_loss_body(enc_ref, w_ref, b_ref, y_ref, out_ref):
    i = pl.program_id(0)
    enc = enc_ref[...]
    w = w_ref[...]
    logits = lax.dot_general(
        enc, w, (((1,), (1,)), ((), ())),
        preferred_element_type=jnp.float32) + b_ref[...]
    m = jnp.max(logits, axis=1, keepdims=True)
    lse = m[:, 0] + jnp.log(jnp.sum(jnp.exp(logits - m), axis=1))
    cols = lax.broadcasted_iota(jnp.int32, (BT, OUT_PAD), 1)
    picked = jnp.sum(jnp.where(cols == y_ref[...], logits, 0.0), axis=1)
    part = jnp.sum(lse - picked) * (1.0 / B)

    @pl.when(i == 0)
    def _():
        out_ref[0, 0] = 0.0

    out_ref[0, 0] += part


def _tc_loss(enc, w_pad, b_pad, y2d):
    out = pl.pallas_call(
        _tc_loss_body,
        grid=(B // BT,),
        in_specs=[
            pl.BlockSpec((BT, EMB), lambda i: (i, 0)),
            pl.BlockSpec((OUT_PAD, EMB), lambda i: (0, 0)),
            pl.BlockSpec((1, OUT_PAD), lambda i: (0, 0)),
            pl.BlockSpec((BT, 1), lambda i: (i, 0)),
        ],
        out_specs=pl.BlockSpec((1, 1), lambda i: (0, 0)),
        out_shape=jax.ShapeDtypeStruct((1, 1), jnp.float32),
    )(enc, w_pad, b_pad, y2d)
    return out[0, 0]


def kernel(input_x, input_y, table, W, b):
    idx = input_x.astype(jnp.int32).reshape(NW, CHUNKS_PER_W, IDX_PER_CHUNK)
    enc = _sc_pool(idx, table)
    w_pad = jnp.zeros((OUT_PAD, EMB), jnp.float32).at[:OUT, :].set(W)
    b_pad = jnp.full((1, OUT_PAD), NEG, jnp.float32).at[0, :OUT].set(b)
    y2d = input_y.astype(jnp.int32).reshape(B, 1)
    return _tc_loss(enc, w_pad, b_pad, y2d)


# speedup vs baseline: 2.9030x; 2.9030x over previous
import functools

import jax
import jax.numpy as jnp
from jax import lax
from jax.experimental import pallas as pl
from jax.experimental.pallas import tpu as pltpu
from jax.experimental.pallas import tpu_sc as plsc

VOCAB = 1000000
EMB = 32
OUT = 100
B = 16384
L = 50

NC = 2
NS = 16
NW = NC * NS
SENT_PER_W = B // NW
SENT_PER_CHUNK = 2
IDX_PER_CHUNK = SENT_PER_CHUNK * L
CHUNKS_PER_W = SENT_PER_W // SENT_PER_CHUNK
K = 8
NBUF = 2 * K
NPAIR = CHUNKS_PER_W // NBUF
INV_L = 1.0 / L

@functools.cache
def _build_sc_pool():
    mesh = plsc.VectorSubcoreMesh(core_axis_name="c", subcore_axis_name="s")
    return functools.partial(
        pl.kernel,
        out_type=jax.ShapeDtypeStruct((B, EMB), jnp.float32),
        mesh=mesh,
        scratch_types=[
            pltpu.VMEM((CHUNKS_PER_W, IDX_PER_CHUNK), jnp.int32),
            pltpu.VMEM((NBUF, IDX_PER_CHUNK, EMB), jnp.float32),
            pltpu.VMEM((SENT_PER_W, EMB), jnp.float32),
            pltpu.SemaphoreType.DMA,
            pltpu.SemaphoreType.DMA,
        ],
        compiler_params=pltpu.CompilerParams(use_tc_tiling_on_sc=False),
    )(_sc_pool_body)


def _sc_pool_body(idx_hbm, table_hbm, out_hbm, idx_v, rows_v, out_v, sem0, sem1):
    wid = lax.axis_index("s") * NC + lax.axis_index("c")

    pltpu.sync_copy(idx_hbm.at[wid], idx_v)

    def fire(j, buf, sem):
        pltpu.make_async_copy(
            table_hbm.at[idx_v.at[j]], rows_v.at[buf], sem).start()

    def drain(buf, sem):
        pltpu.make_async_copy(
            table_hbm.at[idx_v.at[0]], rows_v.at[buf], sem).wait()

    def process(buf, sent_base):
        def acc_body(l, carry):
            a0, a1, b0, b1 = carry
            r = 5 * l
            for u in range(5):
                a0 = a0 + rows_v[buf, r + u, pl.ds(0, 16)]
                a1 = a1 + rows_v[buf, r + u, pl.ds(16, 16)]
                b0 = b0 + rows_v[buf, L + r + u, pl.ds(0, 16)]
                b1 = b1 + rows_v[buf, L + r + u, pl.ds(16, 16)]
            return a0, a1, b0, b1

        z = jnp.zeros((16,), jnp.float32)
        a0, a1, b0, b1 = lax.fori_loop(0, L // 5, acc_body, (z, z, z, z))
        out_v[sent_base, pl.ds(0, 16)] = a0 * INV_L
        out_v[sent_base, pl.ds(16, 16)] = a1 * INV_L
        out_v[sent_base + 1, pl.ds(0, 16)] = b0 * INV_L
        out_v[sent_base + 1, pl.ds(16, 16)] = b1 * INV_L

    for b in range(K):
        fire(b, b, sem0)

    def pair_body(pair, _):
        base = pair * NBUF
        for b in range(K):
            fire(base + K + b, K + b, sem1)
        for b in range(K):
            drain(b, sem0)
        for b in range(K):
            process(b, (base + b) * SENT_PER_CHUNK)

        @pl.when(pair < NPAIR - 1)
        def _():
            for b in range(K):
                fire(base + NBUF + b, b, sem0)

        for b in range(K):
            drain(K + b, sem1)
        for b in range(K):
            process(K + b, (base + K + b) * SENT_PER_CHUNK)
        return 0

    lax.fori_loop(0, NPAIR, pair_body, 0)

    pltpu.sync_copy(out_v, out_hbm.at[pl.ds(wid * SENT_PER_W, SENT_PER_W)])


BT = 2048
OUT_PAD = 128
NEG = -1e30


def _tc_loss_body(enc_ref, w_ref, b_ref, y_ref, out_ref):
    i = pl.program_id(0)
    enc = enc_ref[...]
    w = w_ref[...]
    logits = lax.dot_general(
        enc, w, (((1,), (1,)), ((), ())),
        preferred_element_type=jnp.float32) + b_ref[...]
    m = jnp.max(logits, axis=1, keepdims=True)
    lse = m[:, 0] + jnp.log(jnp.sum(jnp.exp(logits - m), axis=1))
    cols = lax.broadcasted_iota(jnp.int32, (BT, OUT_PAD), 1)
    picked = jnp.sum(jnp.where(cols == y_ref[...], logits, 0.0), axis=1)
    part = jnp.sum(lse - picked) * (1.0 / B)

    @pl.when(i == 0)
    def _():
        out_ref[0, 0] = 0.0

    out_ref[0, 0] += part


def _tc_loss(enc, w_pad, b_pad, y2d):
    out = pl.pallas_call(
        _tc_loss_body,
        grid=(B // BT,),
        in_specs=[
            pl.BlockSpec((BT, EMB), lambda i: (i, 0)),
            pl.BlockSpec((OUT_PAD, EMB), lambda i: (0, 0)),
            pl.BlockSpec((1, OUT_PAD), lambda i: (0, 0)),
            pl.BlockSpec((BT, 1), lambda i: (i, 0)),
        ],
        out_specs=pl.BlockSpec((1, 1), lambda i: (0, 0),
                               memory_space=pltpu.SMEM),
        out_shape=jax.ShapeDtypeStruct((1, 1), jnp.float32),
    )(enc, w_pad, b_pad, y2d)
    return out[0, 0]


def kernel(input_x, input_y, table, W, b):
    idx = input_x.astype(jnp.int32).reshape(NW, CHUNKS_PER_W, IDX_PER_CHUNK)
    enc = _build_sc_pool()(idx, table)
    w_pad = jnp.zeros((OUT_PAD, EMB), jnp.float32).at[:OUT, :].set(W)
    b_pad = jnp.full((1, OUT_PAD), NEG, jnp.float32).at[0, :OUT].set(b)
    y2d = input_y.astype(jnp.int32).reshape(B, 1)
    return _tc_loss(enc, w_pad, b_pad, y2d)


# speedup vs baseline: 4.5109x; 1.5539x over previous
import functools

import jax
import jax.numpy as jnp
from jax import lax
from jax.experimental import pallas as pl
from jax.experimental.pallas import tpu as pltpu
from jax.experimental.pallas import tpu_sc as plsc

VOCAB = 1000000
EMB = 32
OUT = 100
B = 16384
L = 50

NC = 2
NS = 16
NW = NC * NS
SENT_PER_W = B // NW
SENT_PER_CHUNK = 2
IDX_PER_CHUNK = SENT_PER_CHUNK * L
CHUNKS_PER_W = SENT_PER_W // SENT_PER_CHUNK
K = 8
NBUF = 2 * K
NPAIR = CHUNKS_PER_W // NBUF
INV_L = 1.0 / L

@functools.cache
def _build_sc_pool():
    mesh = plsc.VectorSubcoreMesh(core_axis_name="c", subcore_axis_name="s")
    return functools.partial(
        pl.kernel,
        out_type=jax.ShapeDtypeStruct((B, EMB), jnp.float32),
        mesh=mesh,
        scratch_types=[
            pltpu.VMEM((CHUNKS_PER_W, IDX_PER_CHUNK), jnp.int32),
            pltpu.VMEM((NBUF, IDX_PER_CHUNK, EMB), jnp.float32),
            pltpu.VMEM((SENT_PER_W, EMB), jnp.float32),
            pltpu.SemaphoreType.DMA,
            pltpu.SemaphoreType.DMA,
        ],
        compiler_params=pltpu.CompilerParams(use_tc_tiling_on_sc=False),
    )(_sc_pool_body)


def _sc_pool_body(idx_hbm, table_hbm, out_hbm, idx_v, rows_v, out_v, sem0, sem1):
    wid = lax.axis_index("s") * NC + lax.axis_index("c")

    pltpu.sync_copy(idx_hbm.at[wid], idx_v)

    def fire(j, buf, sem):
        pltpu.make_async_copy(
            table_hbm.at[idx_v.at[j]], rows_v.at[buf], sem).start()

    def drain(buf, sem):
        pltpu.make_async_copy(
            table_hbm.at[idx_v.at[0]], rows_v.at[buf], sem).wait()

    def process(buf, sent_base):
        def acc_body(l, carry):
            a0, a1, b0, b1 = carry
            r = 5 * l
            for u in range(5):
                a0 = a0 + rows_v[buf, r + u, pl.ds(0, 16)]
                a1 = a1 + rows_v[buf, r + u, pl.ds(16, 16)]
                b0 = b0 + rows_v[buf, L + r + u, pl.ds(0, 16)]
                b1 = b1 + rows_v[buf, L + r + u, pl.ds(16, 16)]
            return a0, a1, b0, b1

        z = jnp.zeros((16,), jnp.float32)
        a0, a1, b0, b1 = lax.fori_loop(0, L // 5, acc_body, (z, z, z, z))
        out_v[sent_base, pl.ds(0, 16)] = a0 * INV_L
        out_v[sent_base, pl.ds(16, 16)] = a1 * INV_L
        out_v[sent_base + 1, pl.ds(0, 16)] = b0 * INV_L
        out_v[sent_base + 1, pl.ds(16, 16)] = b1 * INV_L

    for b in range(K):
        fire(b, b, sem0)

    def pair_body(pair, _):
        base = pair * NBUF
        for b in range(K):
            fire(base + K + b, K + b, sem1)
        for b in range(K):
            drain(b, sem0)
        for b in range(K):
            process(b, (base + b) * SENT_PER_CHUNK)

        @pl.when(pair < NPAIR - 1)
        def _():
            for b in range(K):
                fire(base + NBUF + b, b, sem0)

        for b in range(K):
            drain(K + b, sem1)
        for b in range(K):
            process(K + b, (base + K + b) * SENT_PER_CHUNK)
        return 0

    lax.fori_loop(0, NPAIR, pair_body, 0)

    pltpu.sync_copy(out_v, out_hbm.at[pl.ds(wid * SENT_PER_W, SENT_PER_W)])


SGRP = 262144
VOCAB_PAD = 4 * SGRP
TBLK = 2048
TGRID = SGRP // TBLK


def _tc_transpose_body(x0, x1, x2, x3, out_ref):
    out_ref[...] = jnp.concatenate(
        [x0[...].T, x1[...].T, x2[...].T, x3[...].T], axis=1)


def _tc_transpose(table_t):
    maxb = VOCAB // TBLK
    specs = [
        pl.BlockSpec(
            (EMB, TBLK),
            lambda j, a=a: (0, jnp.minimum(TGRID * a + j, maxb)))
        for a in range(4)
    ]
    return pl.pallas_call(
        _tc_transpose_body,
        grid=(TGRID,),
        in_specs=specs,
        out_specs=pl.BlockSpec((TBLK, 4 * EMB), lambda j: (j, 0)),
        out_shape=jax.ShapeDtypeStruct((SGRP, 4 * EMB), jnp.float32),
    )(table_t, table_t, table_t, table_t)


BT = 2048
OUT_PAD = 128
NEG = -1e30


def _tc_loss_body(enc_ref, w_ref, b_ref, y_ref, out_ref):
    i = pl.program_id(0)
    enc = enc_ref[...]
    w = w_ref[...]
    logits = lax.dot_general(
        enc, w, (((1,), (1,)), ((), ())),
        preferred_element_type=jnp.float32) + b_ref[...]
    m = jnp.max(logits, axis=1, keepdims=True)
    lse = m[:, 0] + jnp.log(jnp.sum(jnp.exp(logits - m), axis=1))
    cols = lax.broadcasted_iota(jnp.int32, (BT, OUT_PAD), 1)
    picked = jnp.sum(jnp.where(cols == y_ref[...], logits, 0.0), axis=1)
    part = jnp.sum(lse - picked) * (1.0 / B)

    @pl.when(i == 0)
    def _():
        out_ref[0, 0] = 0.0

    out_ref[0, 0] += part


def _tc_loss(enc, w_pad, b_pad, y2d):
    out = pl.pallas_call(
        _tc_loss_body,
        grid=(B // BT,),
        in_specs=[
            pl.BlockSpec((BT, EMB), lambda i: (i, 0)),
            pl.BlockSpec((OUT_PAD, EMB), lambda i: (0, 0)),
            pl.BlockSpec((1, OUT_PAD), lambda i: (0, 0)),
            pl.BlockSpec((BT, 1), lambda i: (i, 0)),
        ],
        out_specs=pl.BlockSpec((1, 1), lambda i: (0, 0),
                               memory_space=pltpu.SMEM),
        out_shape=jax.ShapeDtypeStruct((1, 1), jnp.float32),
    )(enc, w_pad, b_pad, y2d)
    return out[0, 0]


def kernel(input_x, input_y, table, W, b):
    t32 = input_x.astype(jnp.int32)
    idx = (4 * (t32 & (SGRP - 1)) + (t32 >> 18)).reshape(
        NW, CHUNKS_PER_W, IDX_PER_CHUNK)
    tbl_lin = _tc_transpose(table.T).reshape(VOCAB_PAD, EMB)
    enc = _build_sc_pool()(idx, tbl_lin)
    w_pad = jnp.zeros((OUT_PAD, EMB), jnp.float32).at[:OUT, :].set(W)
    b_pad = jnp.full((1, OUT_PAD), NEG, jnp.float32).at[0, :OUT].set(b)
    y2d = input_y.astype(jnp.int32).reshape(B, 1)
    return _tc_loss(enc, w_pad, b_pad, y2d)


# speedup vs baseline: 6.7031x; 1.4860x over previous
import functools

import jax
import jax.numpy as jnp
from jax import lax
from jax.experimental import pallas as pl
from jax.experimental.pallas import tpu as pltpu
from jax.experimental.pallas import tpu_sc as plsc

VOCAB = 1000000
EMB = 32
OUT = 100
B = 16384
L = 50

NC = 2
NS = 16
NW = NC * NS
SENT_PER_W = B // NW
SENT_PER_CHUNK = 2
IDX_PER_CHUNK = SENT_PER_CHUNK * L
CHUNKS_PER_W = SENT_PER_W // SENT_PER_CHUNK
K = 8
NBUF = 2 * K
NPAIR = CHUNKS_PER_W // NBUF
INV_L = 1.0 / L

@functools.cache
def _build_sc_pool():
    mesh = plsc.VectorSubcoreMesh(core_axis_name="c", subcore_axis_name="s")
    return functools.partial(
        pl.kernel,
        out_type=jax.ShapeDtypeStruct((B, EMB), jnp.float32),
        mesh=mesh,
        scratch_types=[
            pltpu.VMEM((CHUNKS_PER_W, IDX_PER_CHUNK), jnp.int32),
            pltpu.VMEM((NBUF, IDX_PER_CHUNK, EMB), jnp.float32),
            pltpu.VMEM((SENT_PER_W, EMB), jnp.float32),
            pltpu.SemaphoreType.DMA,
            pltpu.SemaphoreType.DMA,
        ],
        compiler_params=pltpu.CompilerParams(use_tc_tiling_on_sc=False),
    )(_sc_pool_body)


def _sc_pool_body(idx_hbm, table_hbm, out_hbm, idx_v, rows_v, out_v, sem0, sem1):
    wid = lax.axis_index("s") * NC + lax.axis_index("c")

    pltpu.sync_copy(idx_hbm.at[wid], idx_v)

    def fire(j, buf, sem):
        pltpu.make_async_copy(
            table_hbm.at[idx_v.at[j]], rows_v.at[buf], sem).start()

    def drain(buf, sem):
        pltpu.make_async_copy(
            table_hbm.at[idx_v.at[0]], rows_v.at[buf], sem).wait()

    def process(buf, sent_base):
        def acc_body(l, carry):
            a0, a1, b0, b1 = carry
            r = 5 * l
            for u in range(5):
                a0 = a0 + rows_v[buf, r + u, pl.ds(0, 16)]
                a1 = a1 + rows_v[buf, r + u, pl.ds(16, 16)]
                b0 = b0 + rows_v[buf, L + r + u, pl.ds(0, 16)]
                b1 = b1 + rows_v[buf, L + r + u, pl.ds(16, 16)]
            return a0, a1, b0, b1

        z = jnp.zeros((16,), jnp.float32)
        a0, a1, b0, b1 = lax.fori_loop(0, L // 5, acc_body, (z, z, z, z))
        out_v[sent_base, pl.ds(0, 16)] = a0 * INV_L
        out_v[sent_base, pl.ds(16, 16)] = a1 * INV_L
        out_v[sent_base + 1, pl.ds(0, 16)] = b0 * INV_L
        out_v[sent_base + 1, pl.ds(16, 16)] = b1 * INV_L

    for b in range(K):
        fire(b, b, sem0)

    def pair_body(pair, _):
        base = pair * NBUF
        for b in range(K):
            fire(base + K + b, K + b, sem1)
        for b in range(K):
            drain(b, sem0)
        for b in range(K):
            process(b, (base + b) * SENT_PER_CHUNK)

        @pl.when(pair < NPAIR - 1)
        def _():
            for b in range(K):
                fire(base + NBUF + b, b, sem0)

        for b in range(K):
            drain(K + b, sem1)
        for b in range(K):
            process(K + b, (base + K + b) * SENT_PER_CHUNK)
        return 0

    lax.fori_loop(0, NPAIR, pair_body, 0)

    pltpu.sync_copy(out_v, out_hbm.at[pl.ds(wid * SENT_PER_W, SENT_PER_W)])


SGRP = 262144
VOCAB_PAD = 4 * SGRP
TBLK = 2048
TGRID = SGRP // TBLK


def _tc_transpose_body(x0, x1, x2, x3, out_ref):
    x = jnp.concatenate([x0[...], x1[...], x2[...], x3[...]], axis=0)
    eye = jnp.eye(4 * EMB, dtype=jnp.float32)
    out_ref[...] = jax.lax.dot_general(
        x, eye, (((0,), (0,)), ((), ())),
        preferred_element_type=jnp.float32)


def _tc_transpose(table_t):
    maxb = VOCAB // TBLK
    specs = [
        pl.BlockSpec(
            (EMB, TBLK),
            lambda j, a=a: (0, jnp.minimum(TGRID * a + j, maxb)))
        for a in range(4)
    ]
    return pl.pallas_call(
        _tc_transpose_body,
        grid=(TGRID,),
        in_specs=specs,
        out_specs=pl.BlockSpec((TBLK, 4 * EMB), lambda j: (j, 0)),
        out_shape=jax.ShapeDtypeStruct((SGRP, 4 * EMB), jnp.float32),
    )(table_t, table_t, table_t, table_t)


BT = 2048
OUT_PAD = 128
NEG = -1e30


def _tc_loss_body(enc_ref, w_ref, b_ref, y_ref, out_ref):
    i = pl.program_id(0)
    enc = enc_ref[...]
    w = w_ref[...]
    logits = lax.dot_general(
        enc, w, (((1,), (1,)), ((), ())),
        preferred_element_type=jnp.float32) + b_ref[...]
    m = jnp.max(logits, axis=1, keepdims=True)
    lse = m[:, 0] + jnp.log(jnp.sum(jnp.exp(logits - m), axis=1))
    cols = lax.broadcasted_iota(jnp.int32, (BT, OUT_PAD), 1)
    picked = jnp.sum(jnp.where(cols == y_ref[...], logits, 0.0), axis=1)
    part = jnp.sum(lse - picked) * (1.0 / B)

    @pl.when(i == 0)
    def _():
        out_ref[0, 0] = 0.0

    out_ref[0, 0] += part


def _tc_loss(enc, w_pad, b_pad, y2d):
    out = pl.pallas_call(
        _tc_loss_body,
        grid=(B // BT,),
        in_specs=[
            pl.BlockSpec((BT, EMB), lambda i: (i, 0)),
            pl.BlockSpec((OUT_PAD, EMB), lambda i: (0, 0)),
            pl.BlockSpec((1, OUT_PAD), lambda i: (0, 0)),
            pl.BlockSpec((BT, 1), lambda i: (i, 0)),
        ],
        out_specs=pl.BlockSpec((1, 1), lambda i: (0, 0),
                               memory_space=pltpu.SMEM),
        out_shape=jax.ShapeDtypeStruct((1, 1), jnp.float32),
    )(enc, w_pad, b_pad, y2d)
    return out[0, 0]


def kernel(input_x, input_y, table, W, b):
    t32 = input_x.astype(jnp.int32)
    idx = (4 * (t32 & (SGRP - 1)) + (t32 >> 18)).reshape(
        NW, CHUNKS_PER_W, IDX_PER_CHUNK)
    tbl_lin = _tc_transpose(table.T).reshape(VOCAB_PAD, EMB)
    enc = _build_sc_pool()(idx, tbl_lin)
    w_pad = jnp.zeros((OUT_PAD, EMB), jnp.float32).at[:OUT, :].set(W)
    b_pad = jnp.full((1, OUT_PAD), NEG, jnp.float32).at[0, :OUT].set(b)
    y2d = input_y.astype(jnp.int32).reshape(B, 1)
    return _tc_loss(enc, w_pad, b_pad, y2d)


# speedup vs baseline: 8.6382x; 1.2887x over previous
import functools

import jax
import jax.numpy as jnp
from jax import lax
from jax.experimental import pallas as pl
from jax.experimental.pallas import tpu as pltpu
from jax.experimental.pallas import tpu_sc as plsc

VOCAB = 1000000
EMB = 32
OUT = 100
B = 16384
L = 50

NC = 2
NS = 16
NW = NC * NS
SENT_PER_W = B // NW
SENT_PER_CHUNK = 2
IDX_PER_CHUNK = SENT_PER_CHUNK * L
CHUNKS_PER_W = SENT_PER_W // SENT_PER_CHUNK
K = 8
NBUF = 2 * K
NPAIR = CHUNKS_PER_W // NBUF
INV_L = 1.0 / L

@functools.cache
def _build_sc_pool():
    mesh = plsc.VectorSubcoreMesh(core_axis_name="c", subcore_axis_name="s")
    return functools.partial(
        pl.kernel,
        out_type=jax.ShapeDtypeStruct((B, EMB), jnp.float32),
        mesh=mesh,
        scratch_types=[
            pltpu.VMEM((CHUNKS_PER_W, IDX_PER_CHUNK), jnp.int32),
            pltpu.VMEM((NBUF, IDX_PER_CHUNK, EMB), jnp.float32),
            pltpu.VMEM((SENT_PER_W, EMB), jnp.float32),
            pltpu.SemaphoreType.DMA,
            pltpu.SemaphoreType.DMA,
        ],
        compiler_params=pltpu.CompilerParams(use_tc_tiling_on_sc=False),
    )(_sc_pool_body)


def _sc_pool_body(idx_hbm, table_hbm, out_hbm, idx_v, rows_v, out_v, sem0, sem1):
    wid = lax.axis_index("s") * NC + lax.axis_index("c")

    pltpu.sync_copy(idx_hbm.at[wid], idx_v)

    def fire(j, buf, sem):
        pltpu.make_async_copy(
            table_hbm.at[idx_v.at[j]], rows_v.at[buf], sem).start()

    def drain(buf, sem):
        pltpu.make_async_copy(
            table_hbm.at[idx_v.at[0]], rows_v.at[buf], sem).wait()

    def process(buf, sent_base):
        def acc_body(l, carry):
            a0, a1, b0, b1 = carry
            r = 5 * l
            for u in range(5):
                a0 = a0 + rows_v[buf, r + u, pl.ds(0, 16)]
                a1 = a1 + rows_v[buf, r + u, pl.ds(16, 16)]
                b0 = b0 + rows_v[buf, L + r + u, pl.ds(0, 16)]
                b1 = b1 + rows_v[buf, L + r + u, pl.ds(16, 16)]
            return a0, a1, b0, b1

        z = jnp.zeros((16,), jnp.float32)
        a0, a1, b0, b1 = lax.fori_loop(0, L // 5, acc_body, (z, z, z, z))
        out_v[sent_base, pl.ds(0, 16)] = a0 * INV_L
        out_v[sent_base, pl.ds(16, 16)] = a1 * INV_L
        out_v[sent_base + 1, pl.ds(0, 16)] = b0 * INV_L
        out_v[sent_base + 1, pl.ds(16, 16)] = b1 * INV_L

    for b in range(K):
        fire(b, b, sem0)

    def pair_body(pair, _):
        base = pair * NBUF
        for b in range(K):
            fire(base + K + b, K + b, sem1)
        for b in range(K):
            drain(b, sem0)
        for b in range(K):
            process(b, (base + b) * SENT_PER_CHUNK)

        @pl.when(pair < NPAIR - 1)
        def _():
            for b in range(K):
                fire(base + NBUF + b, b, sem0)

        for b in range(K):
            drain(K + b, sem1)
        for b in range(K):
            process(K + b, (base + K + b) * SENT_PER_CHUNK)
        return 0

    lax.fori_loop(0, NPAIR, pair_body, 0)

    pltpu.sync_copy(out_v, out_hbm.at[pl.ds(wid * SENT_PER_W, SENT_PER_W)])


SGRP = 262144
VOCAB_PAD = 4 * SGRP
TBLK = 8192
TGRID = SGRP // TBLK


def _tc_transpose_body(x0, x1, x2, x3, out_ref):
    x = jnp.concatenate([x0[...], x1[...], x2[...], x3[...]], axis=0)
    eye = jnp.eye(4 * EMB, dtype=jnp.float32)
    out_ref[...] = jax.lax.dot_general(
        x, eye, (((0,), (0,)), ((), ())),
        preferred_element_type=jnp.float32)


def _tc_transpose(table_t):
    maxb = VOCAB // TBLK
    specs = [
        pl.BlockSpec(
            (EMB, TBLK),
            lambda j, a=a: (0, jnp.minimum(TGRID * a + j, maxb)))
        for a in range(4)
    ]
    return pl.pallas_call(
        _tc_transpose_body,
        grid=(TGRID,),
        in_specs=specs,
        out_specs=pl.BlockSpec((TBLK, 4 * EMB), lambda j: (j, 0)),
        out_shape=jax.ShapeDtypeStruct((SGRP, 4 * EMB), jnp.float32),
    )(table_t, table_t, table_t, table_t)


BT = 2048
OUT_PAD = 128
NEG = -1e30


def _tc_loss_body(enc_ref, w_ref, b_ref, y_ref, out_ref):
    i = pl.program_id(0)
    enc = enc_ref[...]
    w = w_ref[...]
    logits = lax.dot_general(
        enc, w, (((1,), (1,)), ((), ())),
        preferred_element_type=jnp.float32) + b_ref[...]
    m = jnp.max(logits, axis=1, keepdims=True)
    lse = m[:, 0] + jnp.log(jnp.sum(jnp.exp(logits - m), axis=1))
    cols = lax.broadcasted_iota(jnp.int32, (BT, OUT_PAD), 1)
    picked = jnp.sum(jnp.where(cols == y_ref[...], logits, 0.0), axis=1)
    part = jnp.sum(lse - picked) * (1.0 / B)

    @pl.when(i == 0)
    def _():
        out_ref[0, 0] = 0.0

    out_ref[0, 0] += part


def _tc_loss(enc, w_pad, b_pad, y2d):
    out = pl.pallas_call(
        _tc_loss_body,
        grid=(B // BT,),
        in_specs=[
            pl.BlockSpec((BT, EMB), lambda i: (i, 0)),
            pl.BlockSpec((OUT_PAD, EMB), lambda i: (0, 0)),
            pl.BlockSpec((1, OUT_PAD), lambda i: (0, 0)),
            pl.BlockSpec((BT, 1), lambda i: (i, 0)),
        ],
        out_specs=pl.BlockSpec((1, 1), lambda i: (0, 0),
                               memory_space=pltpu.SMEM),
        out_shape=jax.ShapeDtypeStruct((1, 1), jnp.float32),
    )(enc, w_pad, b_pad, y2d)
    return out[0, 0]


def kernel(input_x, input_y, table, W, b):
    t32 = input_x.astype(jnp.int32)
    idx = (4 * (t32 & (SGRP - 1)) + (t32 >> 18)).reshape(
        NW, CHUNKS_PER_W, IDX_PER_CHUNK)
    tbl_lin = _tc_transpose(table.T).reshape(VOCAB_PAD, EMB)
    enc = _build_sc_pool()(idx, tbl_lin)
    w_pad = jnp.zeros((OUT_PAD, EMB), jnp.float32).at[:OUT, :].set(W)
    b_pad = jnp.full((1, OUT_PAD), NEG, jnp.float32).at[0, :OUT].set(b)
    y2d = input_y.astype(jnp.int32).reshape(B, 1)
    return _tc_loss(enc, w_pad, b_pad, y2d)
